# baseline (speedup 1.0000x reference)
import functools

import jax
import jax.numpy as jnp
from jax import lax
from jax.experimental import pallas as pl
from jax.experimental.pallas import tpu as pltpu
from jax.experimental.pallas import tpu_sc as plsc

_N = 10000
_E = 320000
_HID = 200
_HP = 256
_L = 6
_RB = 2000
_NRB = _N // _RB

_NC = 2
_NS = 16
_RPT = 624
_RPT_LAST = _N - 15 * _RPT
_EPT = _E // _NS
_CH = 184
_NCHUNK = 109
_EPTP = _NCHUNK * _CH
_NQUAD = _NCHUNK // 4
_NAGG = _N + 8


def _agg_body(m_hbm, src2_hbm, dst_hbm, z_hbm, out_hbm,
              spm_agg, sv0, sv1, sv2, sv3, dv0, dv1, dv2, dv3, rows0, rows1,
              gsem0, gsem1, ssem0, ssem1, is0, is1, is2, is3):
    c = lax.axis_index("c")
    s = lax.axis_index("s")
    row0 = pl.multiple_of(s * _RPT, 8)

    @pl.when(s < _NS - 1)
    def _():
        pltpu.sync_copy(z_hbm.at[pl.ds(0, _RPT), :],
                        spm_agg.at[pl.ds(row0, _RPT), :])

    @pl.when(s == _NS - 1)
    def _():
        pltpu.sync_copy(z_hbm, spm_agg.at[pl.ds(15 * _RPT, _RPT_LAST), :])

    plsc.subcore_barrier()
    sbase = c * _NS * _EPTP + s * _EPTP
    dbase = s * _EPTP

    def schunk(q):
        return src2_hbm.at[pl.ds(pl.multiple_of(sbase + q * _CH, 8), _CH)]

    def dchunk(q):
        return dst_hbm.at[pl.ds(pl.multiple_of(dbase + q * _CH, 8), _CH)]

    def pload(q, sv, dv, sem):
        pltpu.async_copy(schunk(q), sv, sem)
        pltpu.async_copy(dchunk(q), dv, sem)

    def pwait(q, sv, dv, sem):
        pltpu.make_async_copy(schunk(q), sv, sem).wait()
        pltpu.make_async_copy(dchunk(q), dv, sem).wait()

    def gather(sv, rows, sem):
        return pltpu.async_copy(m_hbm.at[sv], rows, sem)

    def gwait(sv, rows, sem):
        pltpu.make_async_copy(m_hbm.at[sv], rows, sem).wait()

    def scatter(dv, rows, sem):
        return pltpu.async_copy(rows, spm_agg.at[dv], sem, add=True)

    def swait(dv, rows, sem):
        pltpu.make_async_copy(rows, spm_agg.at[dv], sem).wait()

    pltpu.sync_copy(schunk(0), sv0)
    pltpu.sync_copy(dchunk(0), dv0)
    pload(1, sv1, dv1, is1)
    pload(2, sv2, dv2, is2)
    pload(3, sv3, dv3, is3)
    gather(sv0, rows0, gsem0)

    def body(k, carry):
        q0 = 4 * k
        gwait(sv0, rows0, gsem0)

        @pl.when(k > 0)
        def _():
            swait(dv3, rows1, ssem1)
            pload(q0 + 3, sv3, dv3, is3)

        pwait(q0 + 1, sv1, dv1, is1)
        gather(sv1, rows1, gsem1)
        scatter(dv0, rows0, ssem0)
        swait(dv0, rows0, ssem0)
        pload(q0 + 4, sv0, dv0, is0)
        pwait(q0 + 2, sv2, dv2, is2)
        gwait(sv1, rows1, gsem1)
        gather(sv2, rows0, gsem0)
        scatter(dv1, rows1, ssem1)
        swait(dv1, rows1, ssem1)

        @pl.when(k < _NQUAD - 1)
        def _():
            pload(q0 + 5, sv1, dv1, is1)

        pwait(q0 + 3, sv3, dv3, is3)
        gwait(sv2, rows0, gsem0)
        gather(sv3, rows1, gsem1)
        scatter(dv2, rows0, ssem0)
        swait(dv2, rows0, ssem0)

        @pl.when(k < _NQUAD - 1)
        def _():
            pload(q0 + 6, sv2, dv2, is2)

        pwait(q0 + 4, sv0, dv0, is0)
        gwait(sv3, rows1, gsem1)
        scatter(dv3, rows1, ssem1)
        gather(sv0, rows0, gsem0)
        return carry

    lax.fori_loop(0, _NQUAD, body, 0)
    gwait(sv0, rows0, gsem0)
    scatter(dv0, rows0, ssem0)
    swait(dv3, rows1, ssem1)
    swait(dv0, rows0, ssem0)
    plsc.subcore_barrier()

    @pl.when(s < _NS - 1)
    def _():
        pltpu.sync_copy(spm_agg.at[pl.ds(row0, _RPT), :],
                        out_hbm.at[c, pl.ds(row0, _RPT), :])

    @pl.when(s == _NS - 1)
    def _():
        pltpu.sync_copy(spm_agg.at[pl.ds(15 * _RPT, _RPT_LAST), :],
                        out_hbm.at[c, pl.ds(15 * _RPT, _RPT_LAST), :])


_agg_call = pl.kernel(
    _agg_body,
    out_type=jax.ShapeDtypeStruct((_NC, _N, 128), jnp.float32),
    mesh=plsc.VectorSubcoreMesh(core_axis_name="c", subcore_axis_name="s"),
    scratch_types=[
        pltpu.VMEM_SHARED((_NAGG, 128), jnp.float32),
        pltpu.VMEM((_CH,), jnp.int32),
        pltpu.VMEM((_CH,), jnp.int32),
        pltpu.VMEM((_CH,), jnp.int32),
        pltpu.VMEM((_CH,), jnp.int32),
        pltpu.VMEM((_CH,), jnp.int32),
        pltpu.VMEM((_CH,), jnp.int32),
        pltpu.VMEM((_CH,), jnp.int32),
        pltpu.VMEM((_CH,), jnp.int32),
        pltpu.VMEM((_CH, 128), jnp.float32),
        pltpu.VMEM((_CH, 128), jnp.float32),
        pltpu.SemaphoreType.DMA,
        pltpu.SemaphoreType.DMA,
        pltpu.SemaphoreType.DMA,
        pltpu.SemaphoreType.DMA,
        pltpu.SemaphoreType.DMA,
        pltpu.SemaphoreType.DMA,
        pltpu.SemaphoreType.DMA,
        pltpu.SemaphoreType.DMA,
    ],
)


def _split_slabs(m_ref, mfull):
    m_ref[0] = mfull[:, :128]
    m_ref[1] = mfull[:, 128:]


def _mm_body(h_ref, w_ref, m_ref):
    mfull = jnp.dot(h_ref[...], w_ref[...], preferred_element_type=jnp.float32)
    _split_slabs(m_ref, mfull)


def _matmul(h, w):
    return pl.pallas_call(
        _mm_body,
        grid=(_NRB,),
        in_specs=[pl.BlockSpec((_RB, _HID), lambda i: (i, 0)),
                  pl.BlockSpec((_HID, _HP), lambda i: (0, 0))],
        out_specs=pl.BlockSpec((_NC, _RB, 128), lambda i: (0, i, 0)),
        out_shape=jax.ShapeDtypeStruct((_NC, _N, 128), jnp.float32),
    )(h, w)


def _gru_update(a0, a1, h, wih, whh, bih, bhh):
    agg = jnp.concatenate([a0, a1], axis=1)[:, :_HID]
    gi = jnp.dot(agg, wih, preferred_element_type=jnp.float32) + bih
    gh = jnp.dot(h, whh, preferred_element_type=jnp.float32) + bhh
    r = jax.nn.sigmoid(gi[:, :_HID] + gh[:, :_HID])
    z = jax.nn.sigmoid(gi[:, _HID:2 * _HID] + gh[:, _HID:2 * _HID])
    n = jnp.tanh(gi[:, 2 * _HID:] + r * gh[:, 2 * _HID:])
    return (1.0 - z) * n + z * h


def _gru_mid_body(a0_ref, a1_ref, h_ref, wih_ref, whh_ref, bih_ref, bhh_ref,
                  wn_ref, ho_ref, mo_ref):
    hn = _gru_update(a0_ref[...], a1_ref[...], h_ref[...], wih_ref[...],
                     whh_ref[...], bih_ref[...], bhh_ref[...])
    ho_ref[...] = hn
    mfull = jnp.dot(hn, wn_ref[...], preferred_element_type=jnp.float32)
    _split_slabs(mo_ref, mfull)


def _gru_mid(a0, a1, h, wih, whh, bih, bhh, wn):
    blk = lambda i: (i, 0)
    cst = lambda i: (0, 0)
    return pl.pallas_call(
        _gru_mid_body,
        grid=(_NRB,),
        in_specs=[pl.BlockSpec((_RB, 128), blk),
                  pl.BlockSpec((_RB, 128), blk),
                  pl.BlockSpec((_RB, _HID), blk),
                  pl.BlockSpec((_HID, 3 * _HID), cst),
                  pl.BlockSpec((_HID, 3 * _HID), cst),
                  pl.BlockSpec((1, 3 * _HID), cst),
                  pl.BlockSpec((1, 3 * _HID), cst),
                  pl.BlockSpec((_HID, _HP), cst)],
        out_specs=[pl.BlockSpec((_RB, _HID), blk),
                   pl.BlockSpec((_NC, _RB, 128), lambda i: (0, i, 0))],
        out_shape=[jax.ShapeDtypeStruct((_N, _HID), jnp.float32),
                   jax.ShapeDtypeStruct((_NC, _N, 128), jnp.float32)],
    )(a0, a1, h, wih, whh, bih, bhh, wn)


def _gru_fin_body(a0_ref, a1_ref, h_ref, wih_ref, whh_ref, bih_ref, bhh_ref,
                  cw_ref, cb_ref, res_ref, mx_ref):
    hn = _gru_update(a0_ref[...], a1_ref[...], h_ref[...], wih_ref[...],
                     whh_ref[...], bih_ref[...], bhh_ref[...])
    hn = jnp.maximum(hn, 0.0)
    bm = jnp.max(hn, axis=0, keepdims=True)
    i = pl.program_id(0)

    @pl.when(i == 0)
    def _():
        mx_ref[...] = bm

    @pl.when(i > 0)
    def _():
        mx_ref[...] = jnp.maximum(mx_ref[...], bm)

    res_ref[...] = jax.nn.sigmoid(
        jnp.dot(mx_ref[...], cw_ref[...], preferred_element_type=jnp.float32)
        + cb_ref[...])


def _gru_fin(a0, a1, h, wih, whh, bih, bhh, cw, cb):
    blk = lambda i: (i, 0)
    cst = lambda i: (0, 0)
    return pl.pallas_call(
        _gru_fin_body,
        grid=(_NRB,),
        in_specs=[pl.BlockSpec((_RB, 128), blk),
                  pl.BlockSpec((_RB, 128), blk),
                  pl.BlockSpec((_RB, _HID), blk),
                  pl.BlockSpec((_HID, 3 * _HID), cst),
                  pl.BlockSpec((_HID, 3 * _HID), cst),
                  pl.BlockSpec((1, 3 * _HID), cst),
                  pl.BlockSpec((1, 3 * _HID), cst),
                  pl.BlockSpec((_HID, 2), cst),
                  pl.BlockSpec((1, 2), cst)],
        out_specs=pl.BlockSpec((1, 2), cst),
        out_shape=jax.ShapeDtypeStruct((1, 2), jnp.float32),
        scratch_shapes=[pltpu.VMEM((1, _HID), jnp.float32)],
    )(a0, a1, h, wih, whh, bih, bhh, cw, cb)


def kernel(x, edge_index, weight, w_ih, w_hh, b_ih, b_hh, cls_W, cls_b):
    h = jnp.pad(x, ((0, 0), (0, _HID - x.shape[1])))
    src = edge_index[0]
    dst = edge_index[1]
    pad = _EPTP - _EPT
    src2 = jnp.stack([src, src + _N]).reshape(2, _NS, _EPT)
    src2 = jnp.pad(src2, ((0, 0), (0, 0), (0, pad))).reshape(-1)
    dstp = jnp.pad(dst.reshape(_NS, _EPT), ((0, 0), (0, pad)),
                   constant_values=_N).reshape(-1)
    zeros = jnp.zeros((_RPT_LAST, 128), jnp.float32)
    wpad = jnp.pad(weight, ((0, 0), (0, 0), (0, _HP - _HID)))
    wihT = w_ih.T
    whhT = w_hh.T
    bih = b_ih.reshape(1, -1)
    bhh = b_hh.reshape(1, -1)
    clsWT = cls_W.T
    clsb = cls_b.reshape(1, -1)

    m = _matmul(h, wpad[0])
    for i in range(_L - 1):
        parts = _agg_call(m.reshape(_NC * _N, 128), src2, dstp, zeros)
        h, m = _gru_mid(parts[0], parts[1], h, wihT, whhT, bih, bhh,
                        wpad[i + 1])
    parts = _agg_call(m.reshape(_NC * _N, 128), src2, dstp, zeros)
    result = _gru_fin(parts[0], parts[1], h, wihT, whhT, bih, bhh,
                      clsWT, clsb)
    return (result, x)


# speedup vs baseline: 1.2511x; 1.2511x over previous
import functools

import jax
import jax.numpy as jnp
from jax import lax
from jax.experimental import pallas as pl
from jax.experimental.pallas import tpu as pltpu
from jax.experimental.pallas import tpu_sc as plsc

_N = 10000
_E = 320000
_HID = 200
_HP = 256
_L = 6
_RB = 2000
_NRB = _N // _RB

_NC = 2
_NS = 16
_RPT = 624
_RPT_LAST = _N - 15 * _RPT
_EPT = _E // _NS
_CH = 160
_NCHUNK = _EPT // _CH
_NQUAD = _NCHUNK // 4


def _agg_body(m_hbm, src2_hbm, dst_hbm, z_hbm, out_hbm,
              spm_agg, sv0, sv1, sv2, sv3, dv0, dv1, dv2, dv3, rows0, rows1,
              gsem0, gsem1, ssem0, ssem1, is0, is1, is2, is3):
    c = lax.axis_index("c")
    s = lax.axis_index("s")
    row0 = pl.multiple_of(s * _RPT, 8)

    @pl.when(s < _NS - 1)
    def _():
        pltpu.sync_copy(z_hbm.at[pl.ds(0, _RPT), :],
                        spm_agg.at[pl.ds(row0, _RPT), :])

    @pl.when(s == _NS - 1)
    def _():
        pltpu.sync_copy(z_hbm, spm_agg.at[pl.ds(15 * _RPT, _RPT_LAST), :])

    plsc.subcore_barrier()
    sbase = c * _E + s * _EPT
    dbase = s * _EPT

    def schunk(q):
        return src2_hbm.at[pl.ds(pl.multiple_of(sbase + q * _CH, 8), _CH)]

    def dchunk(q):
        return dst_hbm.at[pl.ds(pl.multiple_of(dbase + q * _CH, 8), _CH)]

    def pload(q, sv, dv, sem):
        pltpu.async_copy(schunk(q), sv, sem)
        pltpu.async_copy(dchunk(q), dv, sem)

    def pwait(q, sv, dv, sem):
        pltpu.make_async_copy(schunk(q), sv, sem).wait()
        pltpu.make_async_copy(dchunk(q), dv, sem).wait()

    def gather(sv, rows, sem):
        return pltpu.async_copy(m_hbm.at[sv], rows, sem)

    def gwait(sv, rows, sem):
        pltpu.make_async_copy(m_hbm.at[sv], rows, sem).wait()

    def scatter(dv, rows, sem):
        return pltpu.async_copy(rows, spm_agg.at[dv], sem, add=True)

    def swait(dv, rows, sem):
        pltpu.make_async_copy(rows, spm_agg.at[dv], sem).wait()

    pltpu.sync_copy(schunk(0), sv0)
    pltpu.sync_copy(dchunk(0), dv0)
    pload(1, sv1, dv1, is1)
    pload(2, sv2, dv2, is2)
    pload(3, sv3, dv3, is3)
    gather(sv0, rows0, gsem0)

    def body(k, carry):
        q0 = 4 * k
        gwait(sv0, rows0, gsem0)

        @pl.when(k > 0)
        def _():
            swait(dv3, rows1, ssem1)
            pload(q0 + 3, sv3, dv3, is3)

        pwait(q0 + 1, sv1, dv1, is1)
        gather(sv1, rows1, gsem1)
        scatter(dv0, rows0, ssem0)
        swait(dv0, rows0, ssem0)
        pload(q0 + 4, sv0, dv0, is0)
        pwait(q0 + 2, sv2, dv2, is2)
        gwait(sv1, rows1, gsem1)
        gather(sv2, rows0, gsem0)
        scatter(dv1, rows1, ssem1)
        swait(dv1, rows1, ssem1)

        @pl.when(k < _NQUAD - 1)
        def _():
            pload(q0 + 5, sv1, dv1, is1)

        pwait(q0 + 3, sv3, dv3, is3)
        gwait(sv2, rows0, gsem0)
        gather(sv3, rows1, gsem1)
        scatter(dv2, rows0, ssem0)
        swait(dv2, rows0, ssem0)

        @pl.when(k < _NQUAD - 1)
        def _():
            pload(q0 + 6, sv2, dv2, is2)

        pwait(q0 + 4, sv0, dv0, is0)
        gwait(sv3, rows1, gsem1)
        scatter(dv3, rows1, ssem1)
        gather(sv0, rows0, gsem0)
        return carry

    lax.fori_loop(0, _NQUAD, body, 0)
    gwait(sv0, rows0, gsem0)
    scatter(dv0, rows0, ssem0)
    swait(dv3, rows1, ssem1)
    swait(dv0, rows0, ssem0)
    plsc.subcore_barrier()

    @pl.when(s < _NS - 1)
    def _():
        pltpu.sync_copy(spm_agg.at[pl.ds(row0, _RPT), :],
                        out_hbm.at[c, pl.ds(row0, _RPT), :])

    @pl.when(s == _NS - 1)
    def _():
        pltpu.sync_copy(spm_agg.at[pl.ds(15 * _RPT, _RPT_LAST), :],
                        out_hbm.at[c, pl.ds(15 * _RPT, _RPT_LAST), :])


_agg_call = pl.kernel(
    _agg_body,
    out_type=jax.ShapeDtypeStruct((_NC, _N, 128), jnp.float32),
    mesh=plsc.VectorSubcoreMesh(core_axis_name="c", subcore_axis_name="s"),
    scratch_types=[
        pltpu.VMEM_SHARED((_N, 128), jnp.float32),
        pltpu.VMEM((_CH,), jnp.int32),
        pltpu.VMEM((_CH,), jnp.int32),
        pltpu.VMEM((_CH,), jnp.int32),
        pltpu.VMEM((_CH,), jnp.int32),
        pltpu.VMEM((_CH,), jnp.int32),
        pltpu.VMEM((_CH,), jnp.int32),
        pltpu.VMEM((_CH,), jnp.int32),
        pltpu.VMEM((_CH,), jnp.int32),
        pltpu.VMEM((_CH, 128), jnp.float32),
        pltpu.VMEM((_CH, 128), jnp.float32),
        pltpu.SemaphoreType.DMA,
        pltpu.SemaphoreType.DMA,
        pltpu.SemaphoreType.DMA,
        pltpu.SemaphoreType.DMA,
        pltpu.SemaphoreType.DMA,
        pltpu.SemaphoreType.DMA,
        pltpu.SemaphoreType.DMA,
        pltpu.SemaphoreType.DMA,
    ],
)


def _split_slabs(m_ref, mfull):
    m_ref[0] = mfull[:, :128]
    m_ref[1] = mfull[:, 128:]


def _mm_body(h_ref, w_ref, m_ref):
    mfull = jnp.dot(h_ref[...], w_ref[...], preferred_element_type=jnp.float32)
    _split_slabs(m_ref, mfull)


def _matmul(h, w):
    return pl.pallas_call(
        _mm_body,
        grid=(_NRB,),
        in_specs=[pl.BlockSpec((_RB, _HID), lambda i: (i, 0)),
                  pl.BlockSpec((_HID, _HP), lambda i: (0, 0))],
        out_specs=pl.BlockSpec((_NC, _RB, 128), lambda i: (0, i, 0)),
        out_shape=jax.ShapeDtypeStruct((_NC, _N, 128), jnp.float32),
    )(h, w)


def _gru_update(a0, a1, h, wih, whh, bih, bhh):
    agg = jnp.concatenate([a0, a1], axis=1)[:, :_HID]
    gi = jnp.dot(agg, wih, preferred_element_type=jnp.float32) + bih
    gh = jnp.dot(h, whh, preferred_element_type=jnp.float32) + bhh
    r = jax.nn.sigmoid(gi[:, :_HID] + gh[:, :_HID])
    z = jax.nn.sigmoid(gi[:, _HID:2 * _HID] + gh[:, _HID:2 * _HID])
    n = jnp.tanh(gi[:, 2 * _HID:] + r * gh[:, 2 * _HID:])
    return (1.0 - z) * n + z * h


def _gru_mid_body(a0_ref, a1_ref, h_ref, wih_ref, whh_ref, bih_ref, bhh_ref,
                  wn_ref, ho_ref, mo_ref):
    hn = _gru_update(a0_ref[...], a1_ref[...], h_ref[...], wih_ref[...],
                     whh_ref[...], bih_ref[...], bhh_ref[...])
    ho_ref[...] = hn
    mfull = jnp.dot(hn, wn_ref[...], preferred_element_type=jnp.float32)
    _split_slabs(mo_ref, mfull)


def _gru_mid(a0, a1, h, wih, whh, bih, bhh, wn):
    blk = lambda i: (i, 0)
    cst = lambda i: (0, 0)
    return pl.pallas_call(
        _gru_mid_body,
        grid=(_NRB,),
        in_specs=[pl.BlockSpec((_RB, 128), blk),
                  pl.BlockSpec((_RB, 128), blk),
                  pl.BlockSpec((_RB, _HID), blk),
                  pl.BlockSpec((_HID, 3 * _HID), cst),
                  pl.BlockSpec((_HID, 3 * _HID), cst),
                  pl.BlockSpec((1, 3 * _HID), cst),
                  pl.BlockSpec((1, 3 * _HID), cst),
                  pl.BlockSpec((_HID, _HP), cst)],
        out_specs=[pl.BlockSpec((_RB, _HID), blk),
                   pl.BlockSpec((_NC, _RB, 128), lambda i: (0, i, 0))],
        out_shape=[jax.ShapeDtypeStruct((_N, _HID), jnp.float32),
                   jax.ShapeDtypeStruct((_NC, _N, 128), jnp.float32)],
    )(a0, a1, h, wih, whh, bih, bhh, wn)


def _gru_fin_body(a0_ref, a1_ref, h_ref, wih_ref, whh_ref, bih_ref, bhh_ref,
                  cw_ref, cb_ref, res_ref, mx_ref):
    hn = _gru_update(a0_ref[...], a1_ref[...], h_ref[...], wih_ref[...],
                     whh_ref[...], bih_ref[...], bhh_ref[...])
    hn = jnp.maximum(hn, 0.0)
    bm = jnp.max(hn, axis=0, keepdims=True)
    i = pl.program_id(0)

    @pl.when(i == 0)
    def _():
        mx_ref[...] = bm

    @pl.when(i > 0)
    def _():
        mx_ref[...] = jnp.maximum(mx_ref[...], bm)

    res_ref[...] = jax.nn.sigmoid(
        jnp.dot(mx_ref[...], cw_ref[...], preferred_element_type=jnp.float32)
        + cb_ref[...])


def _gru_fin(a0, a1, h, wih, whh, bih, bhh, cw, cb):
    blk = lambda i: (i, 0)
    cst = lambda i: (0, 0)
    return pl.pallas_call(
        _gru_fin_body,
        grid=(_NRB,),
        in_specs=[pl.BlockSpec((_RB, 128), blk),
                  pl.BlockSpec((_RB, 128), blk),
                  pl.BlockSpec((_RB, _HID), blk),
                  pl.BlockSpec((_HID, 3 * _HID), cst),
                  pl.BlockSpec((_HID, 3 * _HID), cst),
                  pl.BlockSpec((1, 3 * _HID), cst),
                  pl.BlockSpec((1, 3 * _HID), cst),
                  pl.BlockSpec((_HID, 2), cst),
                  pl.BlockSpec((1, 2), cst)],
        out_specs=pl.BlockSpec((1, 2), cst),
        out_shape=jax.ShapeDtypeStruct((1, 2), jnp.float32),
        scratch_shapes=[pltpu.VMEM((1, _HID), jnp.float32)],
    )(a0, a1, h, wih, whh, bih, bhh, cw, cb)


def kernel(x, edge_index, weight, w_ih, w_hh, b_ih, b_hh, cls_W, cls_b):
    h = jnp.pad(x, ((0, 0), (0, _HID - x.shape[1])))
    src = edge_index[0]
    dst = edge_index[1]
    src2 = jnp.concatenate([src, src + _N])
    zeros = jnp.zeros((_RPT_LAST, 128), jnp.float32)
    wpad = jnp.pad(weight, ((0, 0), (0, 0), (0, _HP - _HID)))
    wihT = w_ih.T
    whhT = w_hh.T
    bih = b_ih.reshape(1, -1)
    bhh = b_hh.reshape(1, -1)
    clsWT = cls_W.T
    clsb = cls_b.reshape(1, -1)

    m = _matmul(h, wpad[0])
    for i in range(_L - 1):
        parts = _agg_call(m.reshape(_NC * _N, 128), src2, dst, zeros)
        h, m = _gru_mid(parts[0], parts[1], h, wihT, whhT, bih, bhh,
                        wpad[i + 1])
    parts = _agg_call(m.reshape(_NC * _N, 128), src2, dst, zeros)
    result = _gru_fin(parts[0], parts[1], h, wihT, whhT, bih, bhh,
                      clsWT, clsb)
    return (result, x)
